# Initial kernel scaffold; baseline (speedup 1.0000x reference)
import jax
import jax.numpy as jnp
from jax.experimental import pallas as pl


def kernel(x, W_in, b_in, codebooks, W_out, b_out):
    raise NotImplementedError("write your pallas kernel here")


# speedup vs baseline: 2.1343x; 2.1343x over previous
import jax
import jax.numpy as jnp
from jax.experimental import pallas as pl
from jax.experimental.pallas import tpu as pltpu

N_Q = 9
CD = 8
K = 1024
D = 512
TT = 512


def _rvq_body(x_ref, wint_ref, bin_ref, cbnt_ref, cbsq_ref, cb_ref,
              woutt_ref, bout_ref, o_ref):
    res = x_ref[0].T
    acc = jnp.zeros((TT, D), jnp.float32)
    for i in range(N_Q):
        z_e = jnp.dot(res, wint_ref[i]) + bin_ref[i]
        nrm = jnp.sqrt(jnp.sum(z_e * z_e, axis=1, keepdims=True))
        enc = z_e / jnp.maximum(nrm, 1e-12)
        mm = jnp.dot(enc, cbnt_ref[i])
        enc_sq = jnp.sum(enc * enc, axis=1, keepdims=True)
        neg_dist = -((enc_sq - 2.0 * mm) + cbsq_ref[i])
        m = jnp.max(neg_dist, axis=1, keepdims=True)
        iota = jax.lax.broadcasted_iota(jnp.int32, (TT, K), 1)
        idx = jnp.min(jnp.where(neg_dist == m, iota, K), axis=1,
                      keepdims=True)
        onehot = (iota == idx).astype(jnp.float32)
        q = jnp.dot(onehot, cb_ref[i])
        out_i = jnp.dot(q, woutt_ref[i]) + bout_ref[i]
        acc = acc + out_i
        if i < N_Q - 1:
            res = res - out_i
    o_ref[0] = acc.T


@jax.jit
def kernel(x, W_in, b_in, codebooks, W_out, b_out):
    b, d, t = x.shape
    n_q, cd, _ = W_in.shape
    k = codebooks.shape[1]

    wint = jnp.transpose(W_in, (0, 2, 1))
    bin_row = b_in[:, None, :]
    nrm_cb = jnp.sqrt(jnp.sum(codebooks * codebooks, axis=-1, keepdims=True))
    cbn = codebooks / jnp.maximum(nrm_cb, 1e-12)
    cbnt = jnp.transpose(cbn, (0, 2, 1))
    cbsq = jnp.sum(cbn * cbn, axis=-1)[:, None, :]
    woutt = jnp.transpose(W_out, (0, 2, 1))
    bout_row = b_out[:, None, :]

    grid = (b, t // TT)
    wspec = lambda shape: pl.BlockSpec(shape, lambda bi, ti: (0,) * len(shape))
    return pl.pallas_call(
        _rvq_body,
        grid=grid,
        in_specs=[
            pl.BlockSpec((1, d, TT), lambda bi, ti: (bi, 0, ti)),
            wspec((n_q, d, cd)),
            wspec((n_q, 1, cd)),
            wspec((n_q, cd, k)),
            wspec((n_q, 1, k)),
            wspec((n_q, k, cd)),
            wspec((n_q, cd, d)),
            wspec((n_q, 1, d)),
        ],
        out_specs=pl.BlockSpec((1, d, TT), lambda bi, ti: (bi, 0, ti)),
        out_shape=jax.ShapeDtypeStruct((b, d, t), jnp.float32),
        compiler_params=pltpu.CompilerParams(
            dimension_semantics=("parallel", "parallel")),
    )(x, wint, bin_row, cbnt, cbsq, codebooks, woutt, bout_row)


# speedup vs baseline: 2.6676x; 1.2499x over previous
import jax
import jax.numpy as jnp
from jax.experimental import pallas as pl
from jax.experimental.pallas import tpu as pltpu

N_Q = 9
CD = 8
K = 1024
D = 512
TT = 512


def _rvq_body(x_ref, wint_ref, bin_ref, cbnt2_ref, cbsq_ref, cbp_ref,
              woutt_ref, bout_ref, o_ref):
    res = x_ref[0].T
    for i in range(N_Q):
        z_e = jnp.dot(res, wint_ref[i]) + bin_ref[i]
        nrm = jnp.sqrt(jnp.sum(z_e * z_e, axis=1, keepdims=True))
        enc = z_e / jnp.maximum(nrm, 1e-12)
        mm2 = jnp.dot(enc, cbnt2_ref[i])
        enc_sq = jnp.sum(enc * enc, axis=1, keepdims=True)
        dist = (enc_sq - mm2) + cbsq_ref[i]
        mn = jnp.min(dist, axis=1, keepdims=True)
        sel = (dist == mn).astype(jnp.float32)
        q9 = jnp.dot(sel, cbp_ref[i])
        q = q9[:, :CD] / q9[:, CD:CD + 1]
        out_i = jnp.dot(q, woutt_ref[i]) + bout_ref[i]
        res = res - out_i
    o_ref[0] = x_ref[0] - res.T


@jax.jit
def kernel(x, W_in, b_in, codebooks, W_out, b_out):
    b, d, t = x.shape
    n_q, cd, _ = W_in.shape
    k = codebooks.shape[1]

    wint = jnp.transpose(W_in, (0, 2, 1))
    bin_row = b_in[:, None, :]
    nrm_cb = jnp.sqrt(jnp.sum(codebooks * codebooks, axis=-1, keepdims=True))
    cbn = codebooks / jnp.maximum(nrm_cb, 1e-12)
    cbnt2 = 2.0 * jnp.transpose(cbn, (0, 2, 1))
    cbsq = jnp.sum(cbn * cbn, axis=-1)[:, None, :]
    cbp = jnp.concatenate(
        [codebooks, jnp.ones((n_q, k, 1), jnp.float32)], axis=-1)
    woutt = jnp.transpose(W_out, (0, 2, 1))
    bout_row = b_out[:, None, :]

    grid = (b, t // TT)
    wspec = lambda shape: pl.BlockSpec(shape, lambda bi, ti: (0,) * len(shape))
    return pl.pallas_call(
        _rvq_body,
        grid=grid,
        in_specs=[
            pl.BlockSpec((1, d, TT), lambda bi, ti: (bi, 0, ti)),
            wspec((n_q, d, cd)),
            wspec((n_q, 1, cd)),
            wspec((n_q, cd, k)),
            wspec((n_q, 1, k)),
            wspec((n_q, k, cd + 1)),
            wspec((n_q, cd, d)),
            wspec((n_q, 1, d)),
        ],
        out_specs=pl.BlockSpec((1, d, TT), lambda bi, ti: (bi, 0, ti)),
        out_shape=jax.ShapeDtypeStruct((b, d, t), jnp.float32),
        compiler_params=pltpu.CompilerParams(
            dimension_semantics=("parallel", "parallel")),
    )(x, wint, bin_row, cbnt2, cbsq, cbp, woutt, bout_row)


# speedup vs baseline: 4.3583x; 1.6338x over previous
import jax
import jax.numpy as jnp
from jax.experimental import pallas as pl
from jax.experimental.pallas import tpu as pltpu

N_Q = 9
CD = 8
K = 1024
D = 512
TT = 2048
SUB = 4


def _rvq_body(x_ref, wint_ref, bin_ref, cbnt2_ref, cbsq_ref, cbp_ref,
              woutt_ref, bout_ref, o_ref):
    h = TT // SUB
    tiles = [x_ref[0, :, j * h:(j + 1) * h].T for j in range(SUB)]
    for i in range(N_Q):
        z_e = [jnp.dot(r, wint_ref[i]) + bin_ref[i] for r in tiles]
        nrm = [jnp.sqrt(jnp.sum(z * z, axis=1, keepdims=True)) for z in z_e]
        enc = [z / jnp.maximum(n, 1e-12) for z, n in zip(z_e, nrm)]
        mm2 = [jnp.dot(e, cbnt2_ref[i]) for e in enc]
        enc_sq = [jnp.sum(e * e, axis=1, keepdims=True) for e in enc]
        dist = [(s - m) + cbsq_ref[i] for s, m in zip(enc_sq, mm2)]
        mn = [jnp.min(dd, axis=1, keepdims=True) for dd in dist]
        sel = [(dd == m).astype(jnp.float32) for dd, m in zip(dist, mn)]
        q9 = [jnp.dot(ss, cbp_ref[i]) for ss in sel]
        q = [qq[:, :CD] / qq[:, CD:CD + 1] for qq in q9]
        out_i = [jnp.dot(qq, woutt_ref[i]) + bout_ref[i] for qq in q]
        tiles = [r - o for r, o in zip(tiles, out_i)]
    res = jnp.concatenate([r.T for r in tiles], axis=1)
    o_ref[0] = x_ref[0] - res


@jax.jit
def kernel(x, W_in, b_in, codebooks, W_out, b_out):
    b, d, t = x.shape
    n_q, cd, _ = W_in.shape
    k = codebooks.shape[1]

    wint = jnp.transpose(W_in, (0, 2, 1))
    bin_row = b_in[:, None, :]
    nrm_cb = jnp.sqrt(jnp.sum(codebooks * codebooks, axis=-1, keepdims=True))
    cbn = codebooks / jnp.maximum(nrm_cb, 1e-12)
    cbnt2 = 2.0 * jnp.transpose(cbn, (0, 2, 1))
    cbsq = jnp.sum(cbn * cbn, axis=-1)[:, None, :]
    cbp = jnp.concatenate(
        [codebooks, jnp.ones((n_q, k, 1), jnp.float32)], axis=-1)
    woutt = jnp.transpose(W_out, (0, 2, 1))
    bout_row = b_out[:, None, :]

    grid = (b, t // TT)
    wspec = lambda shape: pl.BlockSpec(shape, lambda bi, ti: (0,) * len(shape))
    return pl.pallas_call(
        _rvq_body,
        grid=grid,
        in_specs=[
            pl.BlockSpec((1, d, TT), lambda bi, ti: (bi, 0, ti)),
            wspec((n_q, d, cd)),
            wspec((n_q, 1, cd)),
            wspec((n_q, cd, k)),
            wspec((n_q, 1, k)),
            wspec((n_q, k, cd + 1)),
            wspec((n_q, cd, d)),
            wspec((n_q, 1, d)),
        ],
        out_specs=pl.BlockSpec((1, d, TT), lambda bi, ti: (bi, 0, ti)),
        out_shape=jax.ShapeDtypeStruct((b, d, t), jnp.float32),
        compiler_params=pltpu.CompilerParams(
            dimension_semantics=("parallel", "parallel")),
    )(x, wint, bin_row, cbnt2, cbsq, cbp, woutt, bout_row)


# speedup vs baseline: 5.6609x; 1.2989x over previous
import jax
import jax.numpy as jnp
from jax.experimental import pallas as pl
from jax.experimental.pallas import tpu as pltpu

N_Q = 9
CD = 8
K = 1024
D = 512
TT = 2048
SUB = 4


def _rvq_body(x_ref, win_ref, bin_ref, cbn2_ref, cbsq_ref, cbpt_ref,
              wout_ref, bout_ref, o_ref):
    h = TT // SUB
    tiles = [x_ref[0, :, j * h:(j + 1) * h] for j in range(SUB)]
    for i in range(N_Q):
        z_e = [jnp.dot(win_ref[i], r) + bin_ref[i] for r in tiles]
        nrm = [jnp.sqrt(jnp.sum(z * z, axis=0, keepdims=True)) for z in z_e]
        enc = [z / jnp.maximum(n, 1e-12) for z, n in zip(z_e, nrm)]
        mm2 = [jnp.dot(cbn2_ref[i], e) for e in enc]
        u = [m - cbsq_ref[i] for m in mm2]
        mx = [jnp.max(uu, axis=0, keepdims=True) for uu in u]
        sel = [(uu == m).astype(jnp.float32) for uu, m in zip(u, mx)]
        q9 = [jnp.dot(cbpt_ref[i], ss) for ss in sel]
        q = [qq[:CD, :] / qq[CD:CD + 1, :] for qq in q9]
        out_i = [jnp.dot(wout_ref[i], qq) + bout_ref[i] for qq in q]
        tiles = [r - o for r, o in zip(tiles, out_i)]
    o_ref[0] = x_ref[0] - jnp.concatenate(tiles, axis=1)


@jax.jit
def kernel(x, W_in, b_in, codebooks, W_out, b_out):
    b, d, t = x.shape
    n_q, cd, _ = W_in.shape
    k = codebooks.shape[1]

    bin_col = b_in[:, :, None]
    nrm_cb = jnp.sqrt(jnp.sum(codebooks * codebooks, axis=-1, keepdims=True))
    cbn = codebooks / jnp.maximum(nrm_cb, 1e-12)
    cbn2 = 2.0 * cbn
    cbsq = jnp.sum(cbn * cbn, axis=-1)[:, :, None]
    cbpt = jnp.concatenate(
        [jnp.transpose(codebooks, (0, 2, 1)),
         jnp.ones((n_q, 1, k), jnp.float32)], axis=1)
    bout_col = b_out[:, :, None]

    grid = (b, t // TT)
    wspec = lambda shape: pl.BlockSpec(shape, lambda bi, ti: (0,) * len(shape))
    return pl.pallas_call(
        _rvq_body,
        grid=grid,
        in_specs=[
            pl.BlockSpec((1, d, TT), lambda bi, ti: (bi, 0, ti)),
            wspec((n_q, cd, d)),
            wspec((n_q, cd, 1)),
            wspec((n_q, k, cd)),
            wspec((n_q, k, 1)),
            wspec((n_q, cd + 1, k)),
            wspec((n_q, d, cd)),
            wspec((n_q, d, 1)),
        ],
        out_specs=pl.BlockSpec((1, d, TT), lambda bi, ti: (bi, 0, ti)),
        out_shape=jax.ShapeDtypeStruct((b, d, t), jnp.float32),
        compiler_params=pltpu.CompilerParams(
            dimension_semantics=("parallel", "parallel")),
    )(x, W_in, bin_col, cbn2, cbsq, cbpt, W_out, bout_col)
